# baseline (speedup 1.0000x reference)
import functools

import jax
import jax.numpy as jnp
from jax import lax
from jax.experimental import pallas as pl
from jax.experimental.pallas import tpu as pltpu
from jax.experimental.pallas import tpu_sc as plsc

N_COMP = 16
OUT_CH = 32
H = 128
W = 128
N_PTS = 131072
C = N_COMP * OUT_CH

NC = 2
NS = 16
NW = NC * NS
L = 16

P = 16
PTS_PER_W = N_PTS // NW
ROUNDS = PTS_PER_W // P
RPF = 4
SR = ROUNDS // RPF


def _f32(v):
    return jnp.full((L,), v, dtype=jnp.float32)


def _splat(p):
    return jnp.full((L,), p, dtype=jnp.int32)


_GDN = lax.GatherDimensionNumbers(
    offset_dims=(), collapsed_slice_dims=(0,), start_index_map=(0,))


def _bcast_lane(w, sp):
    return lax.gather(w, sp[:, None], _GDN, (1,),
                      mode=lax.GatherScatterMode.PROMISE_IN_BOUNDS)


def _unpack(v):
    lo = lax.bitcast_convert_type(lax.shift_left(v, 16), jnp.float32)
    hi = lax.bitcast_convert_type(v, jnp.float32)
    return lo, hi


def _sc_body(x0_hbm, x1_hbm, x2_hbm, x3_hbm, uv_tab, st_tab, out_hbm,
             xs_v, idx_u0, idx_s0, idx_u1, idx_s1,
             rows_u0, rows_s0, rows_u1, rows_s1, out_v, sem0, sem1, sem_o):
    wid = lax.axis_index("s") * NC + lax.axis_index("c")
    w_base = wid * PTS_PER_W

    pltpu.sync_copy(x0_hbm.at[pl.ds(w_base, PTS_PER_W)],
                    xs_v.at[pl.ds(0, PTS_PER_W)])
    pltpu.sync_copy(x1_hbm.at[pl.ds(w_base, PTS_PER_W)],
                    xs_v.at[pl.ds(PTS_PER_W, PTS_PER_W)])
    pltpu.sync_copy(x2_hbm.at[pl.ds(w_base, PTS_PER_W)],
                    xs_v.at[pl.ds(2 * PTS_PER_W, PTS_PER_W)])
    pltpu.sync_copy(x3_hbm.at[pl.ds(w_base, PTS_PER_W)],
                    xs_v.at[pl.ds(3 * PTS_PER_W, PTS_PER_W)])

    def plane_prep(r, cx, cy, idx_ref):
        gx = xs_v[pl.ds(cx * PTS_PER_W + r * P, L)] * (W - 1.0)
        gy = xs_v[pl.ds(cy * PTS_PER_W + r * P, L)] * (H - 1.0)
        xi = gx.astype(jnp.int32)
        yi = gy.astype(jnp.int32)
        xi = jnp.minimum(jnp.maximum(xi, 0), W - 2)
        yi = jnp.minimum(jnp.maximum(yi, 0), H - 2)
        fx = gx - xi.astype(jnp.float32)
        fy = gy - yi.astype(jnp.float32)
        ib = yi * W + xi
        idx_ref[pl.ds(0, L)] = ib
        idx_ref[pl.ds(P, L)] = ib + 1
        idx_ref[pl.ds(2 * P, L)] = ib + W
        idx_ref[pl.ds(3 * P, L)] = ib + W + 1
        wx0 = 1.0 - fx
        wy0 = 1.0 - fy
        return (wx0 * wy0, fx * wy0, wx0 * fy, fx * fy)

    slots = ((idx_u0, idx_s0, rows_u0, rows_s0, sem0),
             (idx_u1, idx_s1, rows_u1, rows_s1, sem1))

    def fire(r, slot):
        idx_u, idx_s, rows_u, rows_s, sem = slot
        w_uv = plane_prep(r, 0, 1, idx_u)
        w_st = plane_prep(r, 2, 3, idx_s)

        @pl.when(r < ROUNDS)
        def _():
            pltpu.async_copy(uv_tab.at[idx_u], rows_u, sem)
            pltpu.async_copy(st_tab.at[idx_s], rows_s, sem)

        return w_uv + w_st

    w0 = fire(0, slots[0])

    def sr_body(sr, w_carry):
        @pl.when(sr > 0)
        def _():
            pltpu.make_async_copy(
                out_v,
                out_hbm.at[pl.ds(w_base + (sr - 1) * RPF * P, RPF * P)],
                sem_o).wait()

        w_all = w_carry
        for b in range(RPF):
            r = sr * RPF + b
            cur = slots[b % 2]
            w_next = fire(r + 1, slots[(b + 1) % 2])

            idx_u, idx_s, rows_u, rows_s, sem = cur
            pltpu.make_async_copy(uv_tab.at[idx_u], rows_u, sem).wait()
            pltpu.make_async_copy(st_tab.at[idx_s], rows_s, sem).wait()

            def point_body(p, carry2, w_all=w_all, ob=b * P,
                           rows_u=rows_u, rows_s=rows_s):
                sp = _splat(p)
                wts = [_bcast_lane(w, sp) for w in w_all]
                accs = [_f32(0.0) for _ in range(4)]
                for k in range(N_COMP):
                    c0 = k * L
                    a00 = _unpack(rows_u[p, pl.ds(c0, L)])
                    a01 = _unpack(rows_u[P + p, pl.ds(c0, L)])
                    a10 = _unpack(rows_u[2 * P + p, pl.ds(c0, L)])
                    a11 = _unpack(rows_u[3 * P + p, pl.ds(c0, L)])
                    b00 = _unpack(rows_s[p, pl.ds(c0, L)])
                    b01 = _unpack(rows_s[P + p, pl.ds(c0, L)])
                    b10 = _unpack(rows_s[2 * P + p, pl.ds(c0, L)])
                    b11 = _unpack(rows_s[3 * P + p, pl.ds(c0, L)])
                    u0 = (wts[0] * a00[0] + wts[1] * a01[0]
                          + wts[2] * a10[0] + wts[3] * a11[0])
                    s0 = (wts[4] * b00[0] + wts[5] * b01[0]
                          + wts[6] * b10[0] + wts[7] * b11[0])
                    u1 = (wts[0] * a00[1] + wts[1] * a01[1]
                          + wts[2] * a10[1] + wts[3] * a11[1])
                    s1 = (wts[4] * b00[1] + wts[5] * b01[1]
                          + wts[6] * b10[1] + wts[7] * b11[1])
                    par = k & 1
                    accs[par] = accs[par] + u0 * s0
                    accs[2 + par] = accs[2 + par] + u1 * s1
                out_v[ob + p, pl.ds(0, L)] = jnp.maximum(accs[0] + accs[1],
                                                         0.0)
                out_v[ob + p, pl.ds(L, L)] = jnp.maximum(accs[2] + accs[3],
                                                         0.0)
                return carry2

            lax.fori_loop(0, P, point_body, 0, unroll=False)
            w_all = w_next

        pltpu.async_copy(out_v, out_hbm.at[pl.ds(w_base + sr * RPF * P,
                                                 RPF * P)], sem_o)
        return w_all

    lax.fori_loop(0, SR, sr_body, w0, unroll=False)
    pltpu.make_async_copy(
        out_v,
        out_hbm.at[pl.ds(w_base + (SR - 1) * RPF * P, RPF * P)],
        sem_o).wait()


@jax.jit
def _run(x0, x1, x2, x3, uv_tab, st_tab):
    kern = pl.kernel(
        _sc_body,
        out_type=jax.ShapeDtypeStruct((N_PTS, OUT_CH), jnp.float32),
        mesh=plsc.VectorSubcoreMesh(
            core_axis_name="c", subcore_axis_name="s",
            num_cores=NC, num_subcores=NS),
        scratch_types=[
            pltpu.VMEM((4 * PTS_PER_W + P,), jnp.float32),
            pltpu.VMEM((4 * P,), jnp.int32),
            pltpu.VMEM((4 * P,), jnp.int32),
            pltpu.VMEM((4 * P,), jnp.int32),
            pltpu.VMEM((4 * P,), jnp.int32),
            pltpu.VMEM((4 * P, C // 2), jnp.int32),
            pltpu.VMEM((4 * P, C // 2), jnp.int32),
            pltpu.VMEM((4 * P, C // 2), jnp.int32),
            pltpu.VMEM((4 * P, C // 2), jnp.int32),
            pltpu.VMEM((RPF * P, OUT_CH), jnp.float32),
            pltpu.SemaphoreType.DMA,
            pltpu.SemaphoreType.DMA,
            pltpu.SemaphoreType.DMA,
        ],
    )
    return kern(x0, x1, x2, x3, uv_tab, st_tab)


def _prep_table(plane):
    t = plane[0].reshape(N_COMP, 2, L, H * W)
    t = t.transpose(0, 2, 1, 3).reshape(C, H * W)
    t = t.T.astype(jnp.bfloat16).reshape(H * W, C // 2, 2)
    return lax.bitcast_convert_type(t, jnp.int32)


def kernel(x, uv_plane, st_plane):
    xt = x.T
    return _run(xt[0], xt[1], xt[2], xt[3],
                _prep_table(uv_plane), _prep_table(st_plane))


# speedup vs baseline: 1.3219x; 1.3219x over previous
import functools

import jax
import jax.numpy as jnp
from jax import lax
from jax.experimental import pallas as pl
from jax.experimental.pallas import tpu as pltpu
from jax.experimental.pallas import tpu_sc as plsc

N_COMP = 16
OUT_CH = 32
H = 128
W = 128
N_PTS = 131072
C = N_COMP * OUT_CH

NC = 2
NS = 16
NW = NC * NS
L = 16

P = 16
PTS_PER_W = N_PTS // NW
ROUNDS = PTS_PER_W // P
RPF = 4
SR = ROUNDS // RPF


def _f32(v):
    return jnp.full((L,), v, dtype=jnp.float32)


def _splat(p):
    return jnp.full((L,), p, dtype=jnp.int32)


_GDN = lax.GatherDimensionNumbers(
    offset_dims=(), collapsed_slice_dims=(0,), start_index_map=(0,))


def _bcast_lane(w, sp):
    return lax.gather(w, sp[:, None], _GDN, (1,),
                      mode=lax.GatherScatterMode.PROMISE_IN_BOUNDS)


def _unpack(v):
    lo = lax.bitcast_convert_type(lax.shift_left(v, 16), jnp.float32)
    hi = lax.bitcast_convert_type(v, jnp.float32)
    return lo, hi


def _sc_body(x0_hbm, x1_hbm, x2_hbm, x3_hbm, uv_tab, st_tab, out_hbm,
             xs_v, idx_u0, idx_s0, idx_u1, idx_s1,
             rows_u0, rows_s0, rows_u1, rows_s1, out_v, sem0, sem1, sem_o):
    wid = lax.axis_index("s") * NC + lax.axis_index("c")
    w_base = wid * PTS_PER_W

    pltpu.sync_copy(x0_hbm.at[pl.ds(w_base, PTS_PER_W)],
                    xs_v.at[pl.ds(0, PTS_PER_W)])
    pltpu.sync_copy(x1_hbm.at[pl.ds(w_base, PTS_PER_W)],
                    xs_v.at[pl.ds(PTS_PER_W, PTS_PER_W)])
    pltpu.sync_copy(x2_hbm.at[pl.ds(w_base, PTS_PER_W)],
                    xs_v.at[pl.ds(2 * PTS_PER_W, PTS_PER_W)])
    pltpu.sync_copy(x3_hbm.at[pl.ds(w_base, PTS_PER_W)],
                    xs_v.at[pl.ds(3 * PTS_PER_W, PTS_PER_W)])

    def plane_prep(r, cx, cy, idx_ref):
        gx = xs_v[pl.ds(cx * PTS_PER_W + r * P, L)] * (W - 1.0)
        gy = xs_v[pl.ds(cy * PTS_PER_W + r * P, L)] * (H - 1.0)
        xi = gx.astype(jnp.int32)
        yi = gy.astype(jnp.int32)
        xi = jnp.minimum(jnp.maximum(xi, 0), W - 2)
        yi = jnp.minimum(jnp.maximum(yi, 0), H - 2)
        fx = gx - xi.astype(jnp.float32)
        fy = gy - yi.astype(jnp.float32)
        ib = yi * W + xi
        idx_ref[pl.ds(0, L)] = ib
        idx_ref[pl.ds(P, L)] = ib + 1
        idx_ref[pl.ds(2 * P, L)] = ib + W
        idx_ref[pl.ds(3 * P, L)] = ib + W + 1
        wx0 = 1.0 - fx
        wy0 = 1.0 - fy
        return (wx0 * wy0, fx * wy0, wx0 * fy, fx * fy)

    slots = ((idx_u0, idx_s0, rows_u0, rows_s0, sem0),
             (idx_u1, idx_s1, rows_u1, rows_s1, sem1))

    def fire(r, slot):
        idx_u, idx_s, rows_u, rows_s, sem = slot
        w_uv = plane_prep(r, 0, 1, idx_u)
        w_st = plane_prep(r, 2, 3, idx_s)

        @pl.when(r < ROUNDS)
        def _():
            pltpu.async_copy(uv_tab.at[idx_u], rows_u, sem)
            pltpu.async_copy(st_tab.at[idx_s], rows_s, sem)

        return w_uv + w_st

    w0 = fire(0, slots[0])

    def sr_body(sr, w_carry):
        @pl.when(sr > 0)
        def _():
            pltpu.make_async_copy(
                out_v,
                out_hbm.at[pl.ds(w_base + (sr - 1) * RPF * P, RPF * P)],
                sem_o).wait()

        w_all = w_carry
        for b in range(RPF):
            r = sr * RPF + b
            cur = slots[b % 2]
            w_next = fire(r + 1, slots[(b + 1) % 2])

            idx_u, idx_s, rows_u, rows_s, sem = cur
            pltpu.make_async_copy(uv_tab.at[idx_u], rows_u, sem).wait()
            pltpu.make_async_copy(st_tab.at[idx_s], rows_s, sem).wait()

            def point_body(p, carry2, w_all=w_all, ob=b * P,
                           rows_u=rows_u, rows_s=rows_s):
                sp = _splat(p)
                wts = [_bcast_lane(w, sp) for w in w_all]
                accs = [_f32(0.0) for _ in range(4)]
                for k in range(N_COMP):
                    c0 = k * L
                    a00 = _unpack(rows_u[p, pl.ds(c0, L)])
                    a01 = _unpack(rows_u[P + p, pl.ds(c0, L)])
                    a10 = _unpack(rows_u[2 * P + p, pl.ds(c0, L)])
                    a11 = _unpack(rows_u[3 * P + p, pl.ds(c0, L)])
                    b00 = _unpack(rows_s[p, pl.ds(c0, L)])
                    b01 = _unpack(rows_s[P + p, pl.ds(c0, L)])
                    b10 = _unpack(rows_s[2 * P + p, pl.ds(c0, L)])
                    b11 = _unpack(rows_s[3 * P + p, pl.ds(c0, L)])
                    u0 = (wts[0] * a00[0] + wts[1] * a01[0]
                          + wts[2] * a10[0] + wts[3] * a11[0])
                    s0 = (wts[4] * b00[0] + wts[5] * b01[0]
                          + wts[6] * b10[0] + wts[7] * b11[0])
                    u1 = (wts[0] * a00[1] + wts[1] * a01[1]
                          + wts[2] * a10[1] + wts[3] * a11[1])
                    s1 = (wts[4] * b00[1] + wts[5] * b01[1]
                          + wts[6] * b10[1] + wts[7] * b11[1])
                    par = k & 1
                    accs[par] = accs[par] + u0 * s0
                    accs[2 + par] = accs[2 + par] + u1 * s1
                out_v[ob + p, pl.ds(0, L)] = jnp.maximum(accs[0] + accs[1],
                                                         0.0)
                out_v[ob + p, pl.ds(L, L)] = jnp.maximum(accs[2] + accs[3],
                                                         0.0)
                return carry2

            lax.fori_loop(0, P, point_body, 0, unroll=False)
            w_all = w_next

        pltpu.async_copy(out_v, out_hbm.at[pl.ds(w_base + sr * RPF * P,
                                                 RPF * P)], sem_o)
        return w_all

    lax.fori_loop(0, SR, sr_body, w0, unroll=False)
    pltpu.make_async_copy(
        out_v,
        out_hbm.at[pl.ds(w_base + (SR - 1) * RPF * P, RPF * P)],
        sem_o).wait()


@jax.jit
def _run(x0, x1, x2, x3, uv_tab, st_tab):
    kern = pl.kernel(
        _sc_body,
        out_type=jax.ShapeDtypeStruct((N_PTS, OUT_CH), jnp.float32),
        mesh=plsc.VectorSubcoreMesh(
            core_axis_name="c", subcore_axis_name="s",
            num_cores=NC, num_subcores=NS),
        scratch_types=[
            pltpu.VMEM((4 * PTS_PER_W + P,), jnp.float32),
            pltpu.VMEM((4 * P,), jnp.int32),
            pltpu.VMEM((4 * P,), jnp.int32),
            pltpu.VMEM((4 * P,), jnp.int32),
            pltpu.VMEM((4 * P,), jnp.int32),
            pltpu.VMEM((4 * P, C // 2), jnp.int32),
            pltpu.VMEM((4 * P, C // 2), jnp.int32),
            pltpu.VMEM((4 * P, C // 2), jnp.int32),
            pltpu.VMEM((4 * P, C // 2), jnp.int32),
            pltpu.VMEM((RPF * P, OUT_CH), jnp.float32),
            pltpu.SemaphoreType.DMA,
            pltpu.SemaphoreType.DMA,
            pltpu.SemaphoreType.DMA,
        ],
    )
    return kern(x0, x1, x2, x3, uv_tab, st_tab)


def _prep_table(plane):
    t = jnp.transpose(plane[0], (1, 2, 0)).reshape(H * W, C)
    t = t.reshape(H * W, N_COMP, 2, L).transpose(0, 1, 3, 2)
    t = t.astype(jnp.bfloat16).reshape(H * W, C // 2, 2)
    return lax.bitcast_convert_type(t, jnp.int32)


def kernel(x, uv_plane, st_plane):
    xt = x.T
    return _run(xt[0], xt[1], xt[2], xt[3],
                _prep_table(uv_plane), _prep_table(st_plane))


# speedup vs baseline: 1.3914x; 1.0526x over previous
import functools

import jax
import jax.numpy as jnp
from jax import lax
from jax.experimental import pallas as pl
from jax.experimental.pallas import tpu as pltpu
from jax.experimental.pallas import tpu_sc as plsc

N_COMP = 16
OUT_CH = 32
H = 128
W = 128
N_PTS = 131072
C = N_COMP * OUT_CH

NC = 2
NS = 16
NW = NC * NS
L = 16

P = 16
PTS_PER_W = N_PTS // NW
ROUNDS = PTS_PER_W // P
RPF = 4
SR = ROUNDS // RPF


def _f32(v):
    return jnp.full((L,), v, dtype=jnp.float32)


def _splat(p):
    return jnp.full((L,), p, dtype=jnp.int32)


_GDN = lax.GatherDimensionNumbers(
    offset_dims=(), collapsed_slice_dims=(0,), start_index_map=(0,))


def _bcast_lane(w, sp):
    return lax.gather(w, sp[:, None], _GDN, (1,),
                      mode=lax.GatherScatterMode.PROMISE_IN_BOUNDS)


def _unpack(v):
    lo = lax.bitcast_convert_type(lax.shift_left(v, 16), jnp.float32)
    hi = lax.bitcast_convert_type(v, jnp.float32)
    return lo, hi


def _sc_body(x0_hbm, x1_hbm, x2_hbm, x3_hbm, uv_tab, st_tab, out_hbm,
             xs_v, idx_u0, idx_s0, idx_u1, idx_s1,
             rows_u0, rows_s0, rows_u1, rows_s1, out_v, sem0, sem1, sem_o):
    wid = lax.axis_index("s") * NC + lax.axis_index("c")
    w_base = wid * PTS_PER_W

    pltpu.sync_copy(x0_hbm.at[pl.ds(w_base, PTS_PER_W)],
                    xs_v.at[pl.ds(0, PTS_PER_W)])
    pltpu.sync_copy(x1_hbm.at[pl.ds(w_base, PTS_PER_W)],
                    xs_v.at[pl.ds(PTS_PER_W, PTS_PER_W)])
    pltpu.sync_copy(x2_hbm.at[pl.ds(w_base, PTS_PER_W)],
                    xs_v.at[pl.ds(2 * PTS_PER_W, PTS_PER_W)])
    pltpu.sync_copy(x3_hbm.at[pl.ds(w_base, PTS_PER_W)],
                    xs_v.at[pl.ds(3 * PTS_PER_W, PTS_PER_W)])

    def plane_prep(r, cx, cy, idx_ref):
        gx = xs_v[pl.ds(cx * PTS_PER_W + r * P, L)] * (W - 1.0)
        gy = xs_v[pl.ds(cy * PTS_PER_W + r * P, L)] * (H - 1.0)
        xi = gx.astype(jnp.int32)
        yi = gy.astype(jnp.int32)
        xi = jnp.minimum(jnp.maximum(xi, 0), W - 2)
        yi = jnp.minimum(jnp.maximum(yi, 0), H - 2)
        fx = gx - xi.astype(jnp.float32)
        fy = gy - yi.astype(jnp.float32)
        ib = yi * W + xi
        idx_ref[pl.ds(0, L)] = ib
        idx_ref[pl.ds(P, L)] = ib + 1
        idx_ref[pl.ds(2 * P, L)] = ib + W
        idx_ref[pl.ds(3 * P, L)] = ib + W + 1
        wx0 = 1.0 - fx
        wy0 = 1.0 - fy
        return (wx0 * wy0, fx * wy0, wx0 * fy, fx * fy)

    slots = ((idx_u0, idx_s0, rows_u0, rows_s0, sem0),
             (idx_u1, idx_s1, rows_u1, rows_s1, sem1))

    def fire(r, slot):
        idx_u, idx_s, rows_u, rows_s, sem = slot
        w_uv = plane_prep(r, 0, 1, idx_u)
        w_st = plane_prep(r, 2, 3, idx_s)

        @pl.when(r < ROUNDS)
        def _():
            pltpu.async_copy(uv_tab.at[idx_u], rows_u, sem)
            pltpu.async_copy(st_tab.at[idx_s], rows_s, sem)

        return w_uv + w_st

    w0 = fire(0, slots[0])

    def sr_body(sr, w_carry):
        @pl.when(sr > 0)
        def _():
            pltpu.make_async_copy(
                out_v,
                out_hbm.at[pl.ds(w_base + (sr - 1) * RPF * P, RPF * P)],
                sem_o).wait()

        w_all = w_carry
        for b in range(RPF):
            r = sr * RPF + b
            cur = slots[b % 2]
            w_next = fire(r + 1, slots[(b + 1) % 2])

            idx_u, idx_s, rows_u, rows_s, sem = cur
            pltpu.make_async_copy(uv_tab.at[idx_u], rows_u, sem).wait()
            pltpu.make_async_copy(st_tab.at[idx_s], rows_s, sem).wait()

            def point_body(p, carry2, w_all=w_all, ob=b * P,
                           rows_u=rows_u, rows_s=rows_s):
                sp = _splat(p)
                wts = [_bcast_lane(w, sp) for w in w_all]
                accs = [_f32(0.0) for _ in range(4)]
                for k in range(N_COMP):
                    c0 = k * L
                    a00 = _unpack(rows_u[p, pl.ds(c0, L)])
                    a01 = _unpack(rows_u[P + p, pl.ds(c0, L)])
                    a10 = _unpack(rows_u[2 * P + p, pl.ds(c0, L)])
                    a11 = _unpack(rows_u[3 * P + p, pl.ds(c0, L)])
                    b00 = _unpack(rows_s[p, pl.ds(c0, L)])
                    b01 = _unpack(rows_s[P + p, pl.ds(c0, L)])
                    b10 = _unpack(rows_s[2 * P + p, pl.ds(c0, L)])
                    b11 = _unpack(rows_s[3 * P + p, pl.ds(c0, L)])
                    u0 = (wts[0] * a00[0] + wts[1] * a01[0]
                          + wts[2] * a10[0] + wts[3] * a11[0])
                    s0 = (wts[4] * b00[0] + wts[5] * b01[0]
                          + wts[6] * b10[0] + wts[7] * b11[0])
                    u1 = (wts[0] * a00[1] + wts[1] * a01[1]
                          + wts[2] * a10[1] + wts[3] * a11[1])
                    s1 = (wts[4] * b00[1] + wts[5] * b01[1]
                          + wts[6] * b10[1] + wts[7] * b11[1])
                    par = k & 1
                    accs[par] = accs[par] + u0 * s0
                    accs[2 + par] = accs[2 + par] + u1 * s1
                out_v[ob + p, pl.ds(0, L)] = jnp.maximum(accs[0] + accs[1],
                                                         0.0)
                out_v[ob + p, pl.ds(L, L)] = jnp.maximum(accs[2] + accs[3],
                                                         0.0)
                return carry2

            lax.fori_loop(0, P, point_body, 0, unroll=False)
            w_all = w_next

        pltpu.async_copy(out_v, out_hbm.at[pl.ds(w_base + sr * RPF * P,
                                                 RPF * P)], sem_o)
        return w_all

    lax.fori_loop(0, SR, sr_body, w0, unroll=False)
    pltpu.make_async_copy(
        out_v,
        out_hbm.at[pl.ds(w_base + (SR - 1) * RPF * P, RPF * P)],
        sem_o).wait()


@jax.jit
def _run(x0, x1, x2, x3, uv_tab, st_tab):
    kern = pl.kernel(
        _sc_body,
        out_type=jax.ShapeDtypeStruct((N_PTS, OUT_CH), jnp.float32),
        mesh=plsc.VectorSubcoreMesh(
            core_axis_name="c", subcore_axis_name="s",
            num_cores=NC, num_subcores=NS),
        scratch_types=[
            pltpu.VMEM((4 * PTS_PER_W + P,), jnp.float32),
            pltpu.VMEM((4 * P,), jnp.int32),
            pltpu.VMEM((4 * P,), jnp.int32),
            pltpu.VMEM((4 * P,), jnp.int32),
            pltpu.VMEM((4 * P,), jnp.int32),
            pltpu.VMEM((4 * P, C // 2), jnp.int32),
            pltpu.VMEM((4 * P, C // 2), jnp.int32),
            pltpu.VMEM((4 * P, C // 2), jnp.int32),
            pltpu.VMEM((4 * P, C // 2), jnp.int32),
            pltpu.VMEM((RPF * P, OUT_CH), jnp.float32),
            pltpu.SemaphoreType.DMA,
            pltpu.SemaphoreType.DMA,
            pltpu.SemaphoreType.DMA,
        ],
    )
    return kern(x0, x1, x2, x3, uv_tab, st_tab)


def _prep_table(plane):
    t = plane[0].reshape(N_COMP, 2, L, H * W)
    a = lax.bitcast_convert_type(t[:, 0].astype(jnp.bfloat16), jnp.uint16)
    b = lax.bitcast_convert_type(t[:, 1].astype(jnp.bfloat16), jnp.uint16)
    packed = a.astype(jnp.int32) | (b.astype(jnp.int32) << 16)
    return packed.reshape(C // 2, H * W).T


def kernel(x, uv_plane, st_plane):
    xt = x.T
    return _run(xt[0], xt[1], xt[2], xt[3],
                _prep_table(uv_plane), _prep_table(st_plane))
